# baseline (speedup 1.0000x reference)
import functools

import jax
import jax.numpy as jnp
from jax import lax
from jax.experimental import pallas as pl
from jax.experimental.pallas import tpu as pltpu
from jax.experimental.pallas import tpu_sc as plsc

D = 64
B_ = 4096
S_ = 200
V_ = 1000000

NW = 32
NBT = B_ // 128
NST = S_ // 8
LANES = 16

VTILES = V_ // 128
TPS = 2
NSLAB = VTILES // TPS
SPW = -(-NSLAB // NW)
VTAIL = V_ - VTILES * 128

_CP = pltpu.CompilerParams(use_tc_tiling_on_sc=True, needs_layout_passes=False)
_MESH = plsc.VectorSubcoreMesh(core_axis_name="c", subcore_axis_name="s")


def _pos_encoding(seq_len, d_model):
    pos = jnp.arange(0, seq_len, dtype=jnp.float32)[:, None]
    dim = jnp.arange(0, d_model, dtype=jnp.float32)
    result = jnp.zeros((seq_len, d_model), dtype=jnp.float32)
    even = jnp.sin(pos / 10000 ** (dim[0::2] / d_model))
    odd = jnp.cos(pos / 10000 ** (dim[1::2] / d_model))
    result = result.at[:, 0::2].set(even)
    result = result.at[:, 1::2].set(odd)
    return result


def _repack_table(tableT, tailT):
    W = 128 * TPS

    scratch = [
        pltpu.VMEM((D, W), jnp.float32),
        pltpu.VMEM((D, W), jnp.float32),
        pltpu.VMEM((D * TPS, 128), jnp.float32),
        pltpu.VMEM((D * TPS, 128), jnp.float32),
        pltpu.SemaphoreType.DMA,
        pltpu.SemaphoreType.DMA,
        pltpu.SemaphoreType.DMA,
        pltpu.SemaphoreType.DMA,
    ]

    @functools.partial(
        pl.kernel,
        out_type=jax.ShapeDtypeStruct((V_ // 2, 128), jnp.float32),
        mesh=_MESH,
        scratch_types=scratch,
        compiler_params=_CP,
    )
    def run(tt_hbm, tail_hbm, out_hbm, ib0, ib1, ob0, ob1, is0, is1, os0, os1):
        ibufs, obufs = (ib0, ib1), (ob0, ob1)
        isem, osem = (is0, is1), (os0, os1)

        wid = lax.axis_index("c") * 16 + lax.axis_index("s")

        def load(slab, ri):
            src = tt_hbm.at[:, pl.ds(slab * W, W)]
            return pltpu.make_async_copy(src, ibufs[ri], isem[ri])

        def store(slab, ri):
            dst = out_hbm.at[pl.ds(slab * (D * TPS), D * TPS)]
            return pltpu.make_async_copy(obufs[ri], dst, osem[ri])

        load(wid, 0).start()
        load(wid + NW, 1).start()

        @pl.loop(0, SPW, step=2)
        def slab_pair(k0):
            for p in range(2):
                k = k0 + p
                slab = wid + (k * NW)
                ri = p

                @pl.when(slab < NSLAB)
                def _():
                    load(slab, ri).wait()

                    @pl.when(k >= 2)
                    def _():
                        store(slab - 2 * NW, ri).wait()

                    @pl.loop(0, W, step=4)
                    def col(v0):
                        for dv in range(4):
                            v = v0 + dv
                            flat = v * D
                            row = lax.div(flat, 128)
                            colo = lax.rem(flat, 128)
                            for c0 in range(0, D, LANES):
                                rows = c0 + lax.iota(jnp.int32, LANES)
                                cols = jnp.full((LANES,), v, jnp.int32)
                                t = plsc.load_gather(ibufs[ri], [rows, cols])
                                obufs[ri][row, pl.ds(colo + c0, LANES)] = t

                    store(slab, ri).start()
                    nxt = slab + 2 * NW

                    @pl.when(nxt < NSLAB)
                    def _():
                        load(nxt, ri).start()

        for k in (SPW - 2, SPW - 1):
            last = wid + k * NW

            @pl.when(last < NSLAB)
            def _():
                store(last, k % 2).wait()

        @pl.when(wid == 0)
        def _():
            pltpu.sync_copy(tail_hbm, ibufs[0].at[:, pl.ds(0, 128)])

            @pl.loop(0, VTAIL, step=4)
            def tail_col(v0):
                for dv in range(4):
                    v = v0 + dv
                    flat = v * D
                    row = lax.div(flat, 128)
                    colo = lax.rem(flat, 128)
                    for c0 in range(0, D, LANES):
                        rows = c0 + lax.iota(jnp.int32, LANES)
                        cols = jnp.full((LANES,), v, jnp.int32)
                        t = plsc.load_gather(ibufs[0], [rows, cols])
                        obufs[0][row, pl.ds(colo + c0, LANES)] = t

            dst = out_hbm.at[pl.ds(VTILES * D, VTAIL * D // 128)]
            pltpu.sync_copy(obufs[0].at[pl.ds(0, VTAIL * D // 128)], dst)

    return run(tableT, tailT)


def _lookup(tableL, x4, pe):
    scratch = [
        pltpu.VMEM((NST, 1, 8, 128), jnp.int32),
        pltpu.VMEM((2, 128), jnp.int32),
        pltpu.VMEM((S_, 128), jnp.float32),
        pltpu.VMEM((128, 128), jnp.float32),
        pltpu.VMEM((128, 128), jnp.float32),
        pltpu.VMEM((8, 1, 8, 128), jnp.float32),
        pltpu.VMEM((8, 1, 8, 128), jnp.float32),
        pltpu.SemaphoreType.DMA,
        pltpu.SemaphoreType.DMA,
        pltpu.SemaphoreType.DMA,
        pltpu.SemaphoreType.DMA,
    ]

    @functools.partial(
        pl.kernel,
        out_type=jax.ShapeDtypeStruct((S_, 8, NBT, 8, 128), jnp.float32),
        mesh=_MESH,
        scratch_types=scratch,
        compiler_params=_CP,
    )
    def run(tl_hbm, x4_hbm, pe_hbm, out_hbm,
            idx_v, ih_v, pe_v, gb0, gb1, tb0, tb1, gs0, gs1, ss0, ss1):
        gbufs, tbufs = (gb0, gb1), (tb0, tb1)
        gsem, ssem = (gs0, gs1), (ss0, ss1)

        wid = lax.axis_index("c") * 16 + lax.axis_index("s")

        pltpu.sync_copy(x4_hbm.at[:, pl.ds(wid, 1)], idx_v)
        pltpu.sync_copy(pe_hbm, pe_v)

        def gather(gi):
            return pltpu.make_async_copy(tl_hbm.at[ih_v.at[gi]], gbufs[gi],
                                         gsem[gi])

        def halve(s, gi):
            for b0 in range(0, 128, LANES):
                raw = idx_v[s // 8, 0, s % 8, pl.ds(b0, LANES)]
                ih_v[gi, pl.ds(b0, LANES)] = lax.shift_right_logical(raw, 1)

        def store(s, si):
            dst = out_hbm.at[s, pl.ds(0, 8), pl.ds(wid, 1)]
            return pltpu.make_async_copy(tbufs[si], dst, ssem[si])

        halve(0, 0)
        gather(0).start()

        @pl.loop(0, S_, step=2)
        def seq_pair(s0):
            for p in range(2):
                s = s0 + p
                gi = p
                si = p
                gather(gi).wait()

                @pl.when(s + 1 < S_)
                def _():
                    halve(s + 1, 1 - gi)
                    gather(1 - gi).start()

                @pl.when(s >= 2)
                def _():
                    store(s - 2, si).wait()

                for b0 in range(0, 128, LANES):
                    rows16 = b0 + lax.iota(jnp.int32, LANES)
                    raw = idx_v[s // 8, 0, s % 8, pl.ds(b0, LANES)]
                    par64 = lax.shift_left(
                        lax.bitwise_and(raw, jnp.int32(1)), 6)

                    @pl.loop(0, 4)
                    def col_tile(c16):
                        pe_vec = pe_v[s, pl.ds(c16 * LANES, LANES)]
                        for j in range(LANES):
                            c = c16 * LANES + j
                            c8 = c16 * 2 + j // 8
                            cl = j % 8
                            cols = par64 + c
                            v = plsc.load_gather(gbufs[gi], [rows16, cols])
                            tbufs[si][c8, 0, cl, pl.ds(b0, LANES)] = (
                                v + pe_vec[j])

                store(s, si).start()

        store(S_ - 2, 0).wait()
        store(S_ - 1, 1).wait()

    return run(tableL, x4, pe)


@jax.jit
def kernel(x, table):
    tableT = table.T
    x4 = x.T.reshape(NST, 8, NBT, 128).transpose(0, 2, 1, 3)
    pe = jnp.zeros((S_, 128), jnp.float32).at[:, :D].set(_pos_encoding(S_, D))

    tailT = jnp.zeros((D, 128), jnp.float32).at[:, :VTAIL].set(
        table[VTILES * 128:, :].T)
    tableL = _repack_table(tableT, tailT)
    out5 = _lookup(tableL, x4, pe)
    return out5.transpose(2, 4, 0, 1, 3).reshape(B_, S_, D)


# speedup vs baseline: 2.3009x; 2.3009x over previous
import functools

import jax
import jax.numpy as jnp
from jax import lax
from jax.experimental import pallas as pl
from jax.experimental.pallas import tpu as pltpu
from jax.experimental.pallas import tpu_sc as plsc

D = 64
B_ = 4096
S_ = 200

NW = 32
BPW = B_ // NW
CHUNK = 104
STEP = 96
CPW = 2 * BPW
NB = 4
LANES = 16


def _pos_encoding(seq_len, d_model):
    pos = jnp.arange(0, seq_len, dtype=jnp.float32)[:, None]
    dim = jnp.arange(0, d_model, dtype=jnp.float32)
    result = jnp.zeros((seq_len, d_model), dtype=jnp.float32)
    even = jnp.sin(pos / 10000 ** (dim[0::2] / d_model))
    odd = jnp.cos(pos / 10000 ** (dim[1::2] / d_model))
    result = result.at[:, 0::2].set(even)
    result = result.at[:, 1::2].set(odd)
    return result


@jax.jit
def kernel(x, table):
    pe = _pos_encoding(S_, D)

    mesh = plsc.VectorSubcoreMesh(core_axis_name="c", subcore_axis_name="s")

    scratch = [
        pltpu.VMEM((BPW, S_), jnp.int32),
        pltpu.VMEM((S_, D), jnp.float32),
    ]
    for _ in range(NB):
        scratch.append(pltpu.VMEM((CHUNK, D), jnp.float32))
    for _ in range(2 * NB):
        scratch.append(pltpu.SemaphoreType.DMA)

    @functools.partial(
        pl.kernel,
        out_type=jax.ShapeDtypeStruct((B_, S_, D), jnp.float32),
        mesh=mesh,
        scratch_types=scratch,
        compiler_params=pltpu.CompilerParams(use_tc_tiling_on_sc=False),
    )
    def run(x_hbm, table_hbm, pe_hbm, out_hbm, idx_v, pe_v, *rest):
        bufs = rest[:NB]
        gsem = rest[NB:2 * NB]
        ssem = rest[2 * NB:]

        wid = lax.axis_index("c") * 16 + lax.axis_index("s")
        batch0 = wid * BPW

        pltpu.sync_copy(x_hbm.at[pl.ds(batch0, BPW)], idx_v)
        pltpu.sync_copy(pe_hbm, pe_v)

        def issue_gather(g, half, b):
            idx = idx_v.at[g // 2, pl.ds(half * STEP, CHUNK)]
            pltpu.async_copy(table_hbm.at[idx], bufs[b], gsem[b])

        def wait_gather(g, half, b):
            idx = idx_v.at[g // 2, pl.ds(half * STEP, CHUNK)]
            pltpu.make_async_copy(table_hbm.at[idx], bufs[b], gsem[b]).wait()

        def issue_store(g, half, b):
            dst = out_hbm.at[batch0 + g // 2, pl.ds(half * STEP, CHUNK)]
            pltpu.async_copy(bufs[b], dst, ssem[b])

        def wait_store(b):
            dst = out_hbm.at[0, pl.ds(0, CHUNK)]
            pltpu.make_async_copy(bufs[b], dst, ssem[b]).wait()

        for b in range(NB - 1):
            issue_gather(b, b % 2, b)

        @pl.loop(0, CPW, step=NB)
        def chunk_group(g0):
            for b in range(NB):
                g = g0 + b
                half = b % 2
                wait_gather(g, half, b)
                pe_base = half * STEP

                @pl.loop(0, CHUNK)
                def add_pe(r):
                    for c in range(D // LANES):
                        slc = pl.ds(c * LANES, LANES)
                        plsc.addupdate(bufs[b].at[r, slc],
                                       pe_v[pe_base + r, slc])

                issue_store(g, half, b)

                f = g + (NB - 1)
                bf = (b + NB - 1) % NB

                @pl.when(f < CPW)
                def _():
                    @pl.when(g >= 1)
                    def _():
                        wait_store(bf)

                    issue_gather(f, (NB - 1 + b) % 2, bf)

        for b in range(NB):
            wait_store(b)

    return run(x, table, pe)
